# Initial kernel scaffold; baseline (speedup 1.0000x reference)
import jax
import jax.numpy as jnp
from jax import lax
from jax.experimental import pallas as pl
from jax.experimental.pallas import tpu as pltpu
from jax.experimental.pallas import tpu_sc as plsc

N = 10000
E = 160000
D = 128
H = 128
DE = 16
CP = 16
TW = D + CP

NC = 2
NS = 16
NW = NC * NS
EPW = E // NW
CH = 125
NCHUNK = EPW // CH
NPT = N // NS

EBLK = 2000
NBLK = 2000


def _silu(x):
    return x / (1.0 + jnp.exp(-x))


def _pre_body(h_ref, cp_ref, wat_ref, wbt_ref, t1_ref, t2_ref):
    hb = h_ref[...]
    c = cp_ref[...]
    t1_ref[:, :D] = jnp.dot(hb, wat_ref[...], preferred_element_type=jnp.float32)
    t1_ref[:, D:] = c
    t2_ref[:, :D] = jnp.dot(hb, wbt_ref[...], preferred_element_type=jnp.float32)
    t2_ref[:, D:] = -c


def _tc_pre(h, cpad, wat, wbt):
    grid = N // NBLK
    return pl.pallas_call(
        _pre_body,
        grid=(grid,),
        in_specs=[
            pl.BlockSpec((NBLK, D), lambda i: (i, 0)),
            pl.BlockSpec((NBLK, CP), lambda i: (i, 0)),
            pl.BlockSpec((D, D), lambda i: (0, 0)),
            pl.BlockSpec((D, D), lambda i: (0, 0)),
        ],
        out_specs=[
            pl.BlockSpec((NBLK, TW), lambda i: (i, 0)),
            pl.BlockSpec((NBLK, TW), lambda i: (i, 0)),
        ],
        out_shape=[
            jax.ShapeDtypeStruct((N, TW), jnp.float32),
            jax.ShapeDtypeStruct((N, TW), jnp.float32),
        ],
    )(h, cpad, wat, wbt)


def _gather_body(t1_hbm, t2_hbm, row2_hbm, col2_hbm, g_hbm,
                 idxr, idxc, b1, b2, sem):
    c = lax.axis_index("c")
    s = lax.axis_index("s")
    wid = s * NC + c
    pltpu.sync_copy(row2_hbm.at[pl.ds(wid * NCHUNK, NCHUNK)], idxr)
    pltpu.sync_copy(col2_hbm.at[pl.ds(wid * NCHUNK, NCHUNK)], idxc)

    def chunk(j, carry):
        pltpu.async_copy(t1_hbm.at[idxr.at[j]], b1, sem).wait()
        pltpu.async_copy(t2_hbm.at[idxc.at[j]], b2, sem).wait()

        def addrow(r, cc):
            for q in range(TW // 16):
                sl = pl.ds(q * 16, 16)
                b1[r, sl] = b1[r, sl] + b2[r, sl]
            return cc

        lax.fori_loop(0, CH, addrow, 0)
        pltpu.sync_copy(b1, g_hbm.at[pl.ds(wid * EPW + j * CH, CH)])
        return carry

    lax.fori_loop(0, NCHUNK, chunk, 0)


def _sc_gather(t1, t2, row2, col2):
    mesh = plsc.VectorSubcoreMesh(core_axis_name="c", subcore_axis_name="s",
                                  num_cores=NC, num_subcores=NS)
    f = pl.kernel(
        _gather_body,
        out_type=jax.ShapeDtypeStruct((E, TW), jnp.float32),
        mesh=mesh,
        scratch_types=[
            pltpu.VMEM((NCHUNK, CH), jnp.int32),
            pltpu.VMEM((NCHUNK, CH), jnp.int32),
            pltpu.VMEM((CH, TW), jnp.float32),
            pltpu.VMEM((CH, TW), jnp.float32),
            pltpu.SemaphoreType.DMA,
        ],
    )
    return f(t1, t2, row2, col2)


def _edge_body(g_ref, ea_ref, wr_ref, wet_ref, eb1_ref, w2t_ref, eb2_ref,
               c1t_ref, cb1_ref, cw2_ref, e_ref, tr_ref):
    pre = g_ref[:, :D]
    d = g_ref[:, D:]
    radial = jnp.sum(d * d, axis=1, keepdims=True)
    z = (pre + radial * wr_ref[...] + eb1_ref[...]
         + jnp.dot(ea_ref[...], wet_ref[...], preferred_element_type=jnp.float32))
    x = _silu(z)
    e = _silu(jnp.dot(x, w2t_ref[...], preferred_element_type=jnp.float32)
              + eb2_ref[...])
    u = _silu(jnp.dot(e, c1t_ref[...], preferred_element_type=jnp.float32)
              + cb1_ref[...])
    t = jnp.sum(u * cw2_ref[...], axis=1, keepdims=True)
    inv = t / (jnp.sqrt(radial) + 1e-8)
    lane = lax.broadcasted_iota(jnp.int32, d.shape, 1)
    tr = d * inv + jnp.where(lane == 3, 1.0, 0.0)
    e_ref[...] = e
    tr_ref[...] = tr


def _tc_edge(g, edge_attr, wr, wet, eb1, w2t, eb2, c1t, cb1, cw2):
    grid = E // EBLK
    return pl.pallas_call(
        _edge_body,
        grid=(grid,),
        in_specs=[
            pl.BlockSpec((EBLK, TW), lambda i: (i, 0)),
            pl.BlockSpec((EBLK, DE), lambda i: (i, 0)),
            pl.BlockSpec((1, D), lambda i: (0, 0)),
            pl.BlockSpec((DE, D), lambda i: (0, 0)),
            pl.BlockSpec((1, D), lambda i: (0, 0)),
            pl.BlockSpec((D, D), lambda i: (0, 0)),
            pl.BlockSpec((1, D), lambda i: (0, 0)),
            pl.BlockSpec((D, D), lambda i: (0, 0)),
            pl.BlockSpec((1, D), lambda i: (0, 0)),
            pl.BlockSpec((1, D), lambda i: (0, 0)),
        ],
        out_specs=[
            pl.BlockSpec((EBLK, D), lambda i: (i, 0)),
            pl.BlockSpec((EBLK, CP), lambda i: (i, 0)),
        ],
        out_shape=[
            jax.ShapeDtypeStruct((E, D), jnp.float32),
            jax.ShapeDtypeStruct((E, CP), jnp.float32),
        ],
    )(g, edge_attr, wr, wet, eb1, w2t, eb2, c1t, cb1, cw2)


def _scatter_body(e_hbm, tr_hbm, row2_hbm, z128_hbm, z16_hbm,
                  aggp_hbm, ntp_hbm, idxr, ebuf, tbuf, agg_sh, nt_sh):
    c = lax.axis_index("c")
    s = lax.axis_index("s")
    wid = s * NC + c
    pltpu.sync_copy(z128_hbm.at[pl.ds(s * NPT, NPT)],
                    agg_sh.at[pl.ds(s * NPT, NPT)])
    pltpu.sync_copy(z16_hbm.at[pl.ds(s * NPT, NPT)],
                    nt_sh.at[pl.ds(s * NPT, NPT)])
    plsc.subcore_barrier()
    pltpu.sync_copy(row2_hbm.at[pl.ds(wid * NCHUNK, NCHUNK)], idxr)

    def chunk(j, carry):
        pltpu.sync_copy(e_hbm.at[pl.ds(wid * EPW + j * CH, CH)], ebuf)
        pltpu.sync_copy(tr_hbm.at[pl.ds(wid * EPW + j * CH, CH)], tbuf)
        pltpu.sync_copy(ebuf, agg_sh.at[idxr.at[j]], add=True)
        pltpu.sync_copy(tbuf, nt_sh.at[idxr.at[j]], add=True)
        return carry

    lax.fori_loop(0, NCHUNK, chunk, 0)
    plsc.subcore_barrier()
    pltpu.sync_copy(agg_sh.at[pl.ds(s * NPT, NPT)],
                    aggp_hbm.at[c, pl.ds(s * NPT, NPT)])
    pltpu.sync_copy(nt_sh.at[pl.ds(s * NPT, NPT)],
                    ntp_hbm.at[c, pl.ds(s * NPT, NPT)])


def _sc_scatter(e, tr, row2, z128, z16):
    mesh = plsc.VectorSubcoreMesh(core_axis_name="c", subcore_axis_name="s",
                                  num_cores=NC, num_subcores=NS)
    f = pl.kernel(
        _scatter_body,
        out_type=(
            jax.ShapeDtypeStruct((NC, N, D), jnp.float32),
            jax.ShapeDtypeStruct((NC, N, CP), jnp.float32),
        ),
        mesh=mesh,
        scratch_types=[
            pltpu.VMEM((NCHUNK, CH), jnp.int32),
            pltpu.VMEM((CH, D), jnp.float32),
            pltpu.VMEM((CH, CP), jnp.float32),
            pltpu.VMEM_SHARED((N, D), jnp.float32),
            pltpu.VMEM_SHARED((N, CP), jnp.float32),
        ],
    )
    return f(e, tr, row2, z128, z16)


def _node_body(h_ref, ap_ref, nt_ref, cp_ref, w1ht_ref, w1at_ref, nb1_ref,
               w2t_ref, nb2_ref, lnw_ref, lnb_ref, ho_ref, co_ref):
    hb = h_ref[...]
    agg = ap_ref[0] + ap_ref[1]
    st = nt_ref[0] + nt_ref[1]
    x = _silu(jnp.dot(hb, w1ht_ref[...], preferred_element_type=jnp.float32)
              + jnp.dot(agg, w1at_ref[...], preferred_element_type=jnp.float32)
              + nb1_ref[...])
    n = jnp.dot(x, w2t_ref[...], preferred_element_type=jnp.float32) + nb2_ref[...]
    ho = hb + n
    mu = jnp.mean(ho, axis=1, keepdims=True)
    var = jnp.mean((ho - mu) ** 2, axis=1, keepdims=True)
    ho_ref[...] = (ho - mu) / jnp.sqrt(var + 1e-5) * lnw_ref[...] + lnb_ref[...]
    den = st[:, 3:4]
    co_ref[...] = cp_ref[...] + st / (den + 1e-8)


def _tc_node(h, aggp, ntp, cpad, w1ht, w1at, nb1, w2t, nb2, lnw, lnb):
    grid = N // NBLK
    return pl.pallas_call(
        _node_body,
        grid=(grid,),
        in_specs=[
            pl.BlockSpec((NBLK, D), lambda i: (i, 0)),
            pl.BlockSpec((NC, NBLK, D), lambda i: (0, i, 0)),
            pl.BlockSpec((NC, NBLK, CP), lambda i: (0, i, 0)),
            pl.BlockSpec((NBLK, CP), lambda i: (i, 0)),
            pl.BlockSpec((D, D), lambda i: (0, 0)),
            pl.BlockSpec((D, D), lambda i: (0, 0)),
            pl.BlockSpec((1, D), lambda i: (0, 0)),
            pl.BlockSpec((D, D), lambda i: (0, 0)),
            pl.BlockSpec((1, D), lambda i: (0, 0)),
            pl.BlockSpec((1, D), lambda i: (0, 0)),
            pl.BlockSpec((1, D), lambda i: (0, 0)),
        ],
        out_specs=[
            pl.BlockSpec((NBLK, D), lambda i: (i, 0)),
            pl.BlockSpec((NBLK, CP), lambda i: (i, 0)),
        ],
        out_shape=[
            jax.ShapeDtypeStruct((N, D), jnp.float32),
            jax.ShapeDtypeStruct((N, CP), jnp.float32),
        ],
    )(h, aggp, ntp, cpad, w1ht, w1at, nb1, w2t, nb2, lnw, lnb)


def kernel(h, edge_index, coord, edge_attr, ew1, eb1, ew2, eb2,
           nw1, nb1, nw2, nb2, cw1, cb1, cw2, ln_w, ln_b):
    row = edge_index[0].astype(jnp.int32)
    col = edge_index[1].astype(jnp.int32)
    row2 = row.reshape(E // CH, CH)
    col2 = col.reshape(E // CH, CH)
    cpad = jnp.pad(coord, ((0, 0), (0, CP - 3)))

    wat = ew1[:, :D].T
    wbt = ew1[:, D:2 * D].T
    wr = ew1[:, 2 * D].reshape(1, D)
    wet = ew1[:, 2 * D + 1:].T
    w1ht = nw1[:, :D].T
    w1at = nw1[:, D:].T

    t1, t2 = _tc_pre(h, cpad, wat, wbt)
    g = _sc_gather(t1, t2, row2, col2)
    e, tr = _tc_edge(g, edge_attr, wr, wet, eb1.reshape(1, D), ew2.T,
                     eb2.reshape(1, D), cw1.T, cb1.reshape(1, D), cw2)
    z128 = jnp.zeros((N, D), jnp.float32)
    z16 = jnp.zeros((N, CP), jnp.float32)
    aggp, ntp = _sc_scatter(e, tr, row2, z128, z16)
    h_out, co = _tc_node(h, aggp, ntp, cpad, w1ht, w1at, nb1.reshape(1, D),
                         nw2.T, nb2.reshape(1, D), ln_w.reshape(1, D),
                         ln_b.reshape(1, D))
    return (h_out, co[:, :3], e)


# speedup vs baseline: 3.0961x; 3.0961x over previous
import jax
import jax.numpy as jnp
from jax import lax
from jax.experimental import pallas as pl
from jax.experimental.pallas import tpu as pltpu
from jax.experimental.pallas import tpu_sc as plsc

N = 10000
E = 160000
D = 128
H = 128
DE = 16
CP = 16
N8 = 1280

NC = 2
NS = 16
NW = NC * NS
CH = 128
NCHT = E // CH
JMAX = (NCHT + NW - 1) // NW
NPT = 624
NPC = 640
NPT8 = N8 // NS

EBLK = 2000
NBLK = 2000


def _silu(x):
    return x / (1.0 + jnp.exp(-x))


def _pre_body(h_ref, wat_ref, wbt_ref, t1_ref, t2_ref):
    hb = h_ref[...]
    t1_ref[...] = jnp.dot(hb, wat_ref[...], preferred_element_type=jnp.float32)
    t2_ref[...] = jnp.dot(hb, wbt_ref[...], preferred_element_type=jnp.float32)


def _tc_pre(h, wat, wbt):
    grid = N // NBLK
    return pl.pallas_call(
        _pre_body,
        grid=(grid,),
        in_specs=[
            pl.BlockSpec((NBLK, D), lambda i: (i, 0)),
            pl.BlockSpec((D, D), lambda i: (0, 0)),
            pl.BlockSpec((D, D), lambda i: (0, 0)),
        ],
        out_specs=[
            pl.BlockSpec((NBLK, D), lambda i: (i, 0)),
            pl.BlockSpec((NBLK, D), lambda i: (i, 0)),
        ],
        out_shape=[
            jax.ShapeDtypeStruct((N, D), jnp.float32),
            jax.ShapeDtypeStruct((N, D), jnp.float32),
        ],
    )(h, wat, wbt)


def _gather_body(t1_hbm, t2_hbm, row_hbm, col_hbm, c4_hbm, g_hbm, d4_hbm,
                 i1, i2, b1, b2, c4, dbuf, sem):
    c = lax.axis_index("c")
    s = lax.axis_index("s")
    wid = s * NC + c
    pltpu.sync_copy(c4_hbm, c4)
    z16 = jnp.zeros((16,), jnp.float32)

    def zrow(r, cc):
        dbuf[r, :] = z16
        return cc

    lax.fori_loop(0, CH, zrow, 0)
    lane = lax.iota(jnp.int32, 16)

    def chunk(j, carry):
        ch = wid + j * NW

        @pl.when(ch < NCHT)
        def _():
            base = ch * CH
            pltpu.sync_copy(row_hbm.at[pl.ds(base, CH)], i1)
            pltpu.sync_copy(col_hbm.at[pl.ds(base, CH)], i2)
            pltpu.async_copy(t1_hbm.at[i1], b1, sem).wait()
            pltpu.async_copy(t2_hbm.at[i2], b2, sem).wait()

            def addrow(r, cc):
                for q in range(D // 16):
                    sl = pl.ds(q * 16, 16)
                    b1[r, sl] = b1[r, sl] + b2[r, sl]
                return cc

            lax.fori_loop(0, CH, addrow, 0)

            for k in range(CH // 16):
                rows = k * 16 + lane
                ir = i1[pl.ds(k * 16, 16)]
                ic = i2[pl.ds(k * 16, 16)]
                ir4 = ir * 4
                ic4 = ic * 4
                rad = z16
                for q in range(3):
                    cq = jnp.full((16,), q, jnp.int32)
                    xr = plsc.load_gather(c4, [ir4 + q])
                    xc = plsc.load_gather(c4, [ic4 + q])
                    dd = xr - xc
                    rad = rad + dd * dd
                    plsc.store_scatter(dbuf, [rows, cq], dd)
                plsc.store_scatter(dbuf, [rows, jnp.full((16,), 3, jnp.int32)],
                                   rad)
                rem = (ir & 7).astype(jnp.float32)
                plsc.store_scatter(dbuf, [rows, jnp.full((16,), 4, jnp.int32)],
                                   rem)

            pltpu.sync_copy(b1, g_hbm.at[pl.ds(base, CH)])
            pltpu.sync_copy(dbuf, d4_hbm.at[pl.ds(base, CH)])

        return carry

    lax.fori_loop(0, JMAX, chunk, 0)


def _sc_gather(t1, t2, row, col, c4flat):
    mesh = plsc.VectorSubcoreMesh(core_axis_name="c", subcore_axis_name="s",
                                  num_cores=NC, num_subcores=NS)
    f = pl.kernel(
        _gather_body,
        out_type=(
            jax.ShapeDtypeStruct((E, D), jnp.float32),
            jax.ShapeDtypeStruct((E, CP), jnp.float32),
        ),
        mesh=mesh,
        compiler_params=pltpu.CompilerParams(needs_layout_passes=False),
        scratch_types=[
            pltpu.VMEM((CH,), jnp.int32),
            pltpu.VMEM((CH,), jnp.int32),
            pltpu.VMEM((CH, D), jnp.float32),
            pltpu.VMEM((CH, D), jnp.float32),
            pltpu.VMEM((4 * N,), jnp.float32),
            pltpu.VMEM((CH, CP), jnp.float32),
            pltpu.SemaphoreType.DMA,
        ],
    )
    return f(t1, t2, row, col, c4flat)


def _edge_body(g_ref, d4_ref, ea_ref, wr_ref, wet_ref, eb1_ref, w2t_ref,
               eb2_ref, c1t_ref, cb1_ref, cw2_ref, e_ref, pay_ref):
    pre = g_ref[...]
    d4 = d4_ref[...]
    radial = d4[:, 3:4]
    z = (pre + radial * wr_ref[...] + eb1_ref[...]
         + jnp.dot(ea_ref[...], wet_ref[...], preferred_element_type=jnp.float32))
    x = _silu(z)
    e = _silu(jnp.dot(x, w2t_ref[...], preferred_element_type=jnp.float32)
              + eb2_ref[...])
    u = _silu(jnp.dot(e, c1t_ref[...], preferred_element_type=jnp.float32)
              + cb1_ref[...])
    t = jnp.sum(u * cw2_ref[...], axis=1, keepdims=True)
    inv = t / (jnp.sqrt(radial) + 1e-8)
    lane16 = lax.broadcasted_iota(jnp.int32, d4.shape, 1)
    tr = (d4 * inv * jnp.where(lane16 < 3, 1.0, 0.0)
          + jnp.where(lane16 == 3, 1.0, 0.0))
    mi = d4[:, 4:5].astype(jnp.int32)
    lane128 = lax.broadcasted_iota(jnp.int32, pre.shape, 1)
    tr8 = jnp.concatenate([tr] * (D // CP), axis=1)
    pay = jnp.where(lane128 // CP == mi, tr8, 0.0)
    e_ref[...] = e
    pay_ref[...] = pay


def _tc_edge(g, d4, edge_attr, wr, wet, eb1, w2t, eb2, c1t, cb1, cw2):
    grid = E // EBLK
    return pl.pallas_call(
        _edge_body,
        grid=(grid,),
        in_specs=[
            pl.BlockSpec((EBLK, D), lambda i: (i, 0)),
            pl.BlockSpec((EBLK, CP), lambda i: (i, 0)),
            pl.BlockSpec((EBLK, DE), lambda i: (i, 0)),
            pl.BlockSpec((1, D), lambda i: (0, 0)),
            pl.BlockSpec((DE, D), lambda i: (0, 0)),
            pl.BlockSpec((1, D), lambda i: (0, 0)),
            pl.BlockSpec((D, D), lambda i: (0, 0)),
            pl.BlockSpec((1, D), lambda i: (0, 0)),
            pl.BlockSpec((D, D), lambda i: (0, 0)),
            pl.BlockSpec((1, D), lambda i: (0, 0)),
            pl.BlockSpec((1, D), lambda i: (0, 0)),
        ],
        out_specs=[
            pl.BlockSpec((EBLK, D), lambda i: (i, 0)),
            pl.BlockSpec((EBLK, D), lambda i: (i, 0)),
        ],
        out_shape=[
            jax.ShapeDtypeStruct((E, D), jnp.float32),
            jax.ShapeDtypeStruct((E, D), jnp.float32),
        ],
    )(g, d4, edge_attr, wr, wet, eb1, w2t, eb2, c1t, cb1, cw2)


def _scatter_body(e_hbm, pay_hbm, row_hbm, row8_hbm, z_hbm,
                  aggp_hbm, ntp_hbm, i1, i2, ebuf, pbuf, agg_sh, nt_sh):
    c = lax.axis_index("c")
    s = lax.axis_index("s")
    wid = s * NC + c
    nb = s * NPT
    pltpu.sync_copy(z_hbm.at[pl.ds(nb, NPC)], agg_sh.at[pl.ds(nb, NPC)])
    pltpu.sync_copy(z_hbm.at[pl.ds(s * NPT8, NPT8)],
                    nt_sh.at[pl.ds(s * NPT8, NPT8)])
    plsc.subcore_barrier()

    def chunk(j, carry):
        ch = wid + j * NW

        @pl.when(ch < NCHT)
        def _():
            base = ch * CH
            pltpu.sync_copy(row_hbm.at[pl.ds(base, CH)], i1)
            pltpu.sync_copy(row8_hbm.at[pl.ds(base, CH)], i2)
            pltpu.sync_copy(e_hbm.at[pl.ds(base, CH)], ebuf)
            pltpu.sync_copy(pay_hbm.at[pl.ds(base, CH)], pbuf)
            pltpu.sync_copy(ebuf, agg_sh.at[i1], add=True)
            pltpu.sync_copy(pbuf, nt_sh.at[i2], add=True)

        return carry

    lax.fori_loop(0, JMAX, chunk, 0)
    plsc.subcore_barrier()
    pltpu.sync_copy(agg_sh.at[pl.ds(nb, NPC)],
                    aggp_hbm.at[c, pl.ds(nb, NPC)])
    pltpu.sync_copy(nt_sh.at[pl.ds(s * NPT8, NPT8)],
                    ntp_hbm.at[c, pl.ds(s * NPT8, NPT8)])


def _sc_scatter(e, pay, row, row8, z):
    mesh = plsc.VectorSubcoreMesh(core_axis_name="c", subcore_axis_name="s",
                                  num_cores=NC, num_subcores=NS)
    f = pl.kernel(
        _scatter_body,
        out_type=(
            jax.ShapeDtypeStruct((NC, N, D), jnp.float32),
            jax.ShapeDtypeStruct((NC, N8, D), jnp.float32),
        ),
        mesh=mesh,
        scratch_types=[
            pltpu.VMEM((CH,), jnp.int32),
            pltpu.VMEM((CH,), jnp.int32),
            pltpu.VMEM((CH, D), jnp.float32),
            pltpu.VMEM((CH, D), jnp.float32),
            pltpu.VMEM_SHARED((N, D), jnp.float32),
            pltpu.VMEM_SHARED((N8, D), jnp.float32),
        ],
    )
    return f(e, pay, row, row8, z)


def _node_body(h_ref, ap_ref, nt_ref, cp_ref, w1ht_ref, w1at_ref, nb1_ref,
               w2t_ref, nb2_ref, lnw_ref, lnb_ref, ho_ref, co_ref):
    hb = h_ref[...]
    agg = ap_ref[0] + ap_ref[1]
    st = nt_ref[0] + nt_ref[1]
    x = _silu(jnp.dot(hb, w1ht_ref[...], preferred_element_type=jnp.float32)
              + jnp.dot(agg, w1at_ref[...], preferred_element_type=jnp.float32)
              + nb1_ref[...])
    n = jnp.dot(x, w2t_ref[...], preferred_element_type=jnp.float32) + nb2_ref[...]
    ho = hb + n
    mu = jnp.mean(ho, axis=1, keepdims=True)
    var = jnp.mean((ho - mu) ** 2, axis=1, keepdims=True)
    ho_ref[...] = (ho - mu) / jnp.sqrt(var + 1e-5) * lnw_ref[...] + lnb_ref[...]
    den = st[:, 3:4]
    co_ref[...] = cp_ref[...] + st / (den + 1e-8)


def _tc_node(h, aggp, ntp, cpad, w1ht, w1at, nb1, w2t, nb2, lnw, lnb):
    grid = N // NBLK
    return pl.pallas_call(
        _node_body,
        grid=(grid,),
        in_specs=[
            pl.BlockSpec((NBLK, D), lambda i: (i, 0)),
            pl.BlockSpec((NC, NBLK, D), lambda i: (0, i, 0)),
            pl.BlockSpec((NC, NBLK, CP), lambda i: (0, i, 0)),
            pl.BlockSpec((NBLK, CP), lambda i: (i, 0)),
            pl.BlockSpec((D, D), lambda i: (0, 0)),
            pl.BlockSpec((D, D), lambda i: (0, 0)),
            pl.BlockSpec((1, D), lambda i: (0, 0)),
            pl.BlockSpec((D, D), lambda i: (0, 0)),
            pl.BlockSpec((1, D), lambda i: (0, 0)),
            pl.BlockSpec((1, D), lambda i: (0, 0)),
            pl.BlockSpec((1, D), lambda i: (0, 0)),
        ],
        out_specs=[
            pl.BlockSpec((NBLK, D), lambda i: (i, 0)),
            pl.BlockSpec((NBLK, CP), lambda i: (i, 0)),
        ],
        out_shape=[
            jax.ShapeDtypeStruct((N, D), jnp.float32),
            jax.ShapeDtypeStruct((N, CP), jnp.float32),
        ],
    )(h, aggp, ntp, cpad, w1ht, w1at, nb1, w2t, nb2, lnw, lnb)


def kernel(h, edge_index, coord, edge_attr, ew1, eb1, ew2, eb2,
           nw1, nb1, nw2, nb2, cw1, cb1, cw2, ln_w, ln_b):
    row = edge_index[0].astype(jnp.int32)
    col = edge_index[1].astype(jnp.int32)
    row8 = row >> 3
    cpad = jnp.pad(coord, ((0, 0), (0, CP - 3)))
    c4flat = jnp.pad(coord, ((0, 0), (0, 1))).reshape(-1)

    wat = ew1[:, :D].T
    wbt = ew1[:, D:2 * D].T
    wr = ew1[:, 2 * D].reshape(1, D)
    wet = ew1[:, 2 * D + 1:].T
    w1ht = nw1[:, :D].T
    w1at = nw1[:, D:].T

    t1, t2 = _tc_pre(h, wat, wbt)
    g, d4 = _sc_gather(t1, t2, row, col, c4flat)
    e, pay = _tc_edge(g, d4, edge_attr, wr, wet, eb1.reshape(1, D), ew2.T,
                      eb2.reshape(1, D), cw1.T, cb1.reshape(1, D), cw2)
    z = jnp.zeros((N, D), jnp.float32)
    aggp, ntp = _sc_scatter(e, pay, row, row8, z)
    nt = ntp.reshape(NC, N8 * 8, CP)[:, :N, :]
    h_out, co = _tc_node(h, aggp, nt, cpad, w1ht, w1at, nb1.reshape(1, D),
                         nw2.T, nb2.reshape(1, D), ln_w.reshape(1, D),
                         ln_b.reshape(1, D))
    return (h_out, co[:, :3], e)


# speedup vs baseline: 4.0240x; 1.2997x over previous
import jax
import jax.numpy as jnp
from jax import lax
from jax.experimental import pallas as pl
from jax.experimental.pallas import tpu as pltpu
from jax.experimental.pallas import tpu_sc as plsc

N = 10000
E = 160000
D = 128
H = 128
DE = 16
CP = 16
N8 = 1280

NC = 2
NS = 16
NW = NC * NS
NPT = 624
NPC = 640
NPT8 = N8 // NS

CHG = 80
NCHG = E // CHG
JMAXG = (NCHG + NW - 1) // NW
NFULLG = NCHG // NW
EPWG = JMAXG * CHG

CHS = 64
NCHS = E // CHS
JMAXS = (NCHS + NW - 1) // NW
NFULLS = NCHS // NW

EBLK = 2000
NBLK = 2000


def _silu(x):
    return x / (1.0 + jnp.exp(-x))


def _pre_body(h_ref, wat_ref, wbt_ref, t1_ref, t2_ref):
    hb = h_ref[...]
    t1_ref[...] = jnp.dot(hb, wat_ref[...], preferred_element_type=jnp.float32)
    t2_ref[...] = jnp.dot(hb, wbt_ref[...], preferred_element_type=jnp.float32)


def _tc_pre(h, wat, wbt):
    grid = N // NBLK
    return pl.pallas_call(
        _pre_body,
        grid=(grid,),
        in_specs=[
            pl.BlockSpec((NBLK, D), lambda i: (i, 0)),
            pl.BlockSpec((D, D), lambda i: (0, 0)),
            pl.BlockSpec((D, D), lambda i: (0, 0)),
        ],
        out_specs=[
            pl.BlockSpec((NBLK, D), lambda i: (i, 0)),
            pl.BlockSpec((NBLK, D), lambda i: (i, 0)),
        ],
        out_shape=[
            jax.ShapeDtypeStruct((N, D), jnp.float32),
            jax.ShapeDtypeStruct((N, D), jnp.float32),
        ],
    )(h, wat, wbt)


def _gather_body(t1_hbm, t2_hbm, rw_hbm, cw_hbm, c3_hbm, g_hbm, d4_hbm,
                 idxr, idxc, b1a, b2a, dba, b1b, b2b, dbb, c3, semg, semw):
    c = lax.axis_index("c")
    s = lax.axis_index("s")
    wid = s * NC + c
    pltpu.sync_copy(rw_hbm.at[pl.ds(wid * EPWG, EPWG)], idxr)
    pltpu.sync_copy(cw_hbm.at[pl.ds(wid * EPWG, EPWG)], idxc)
    pltpu.sync_copy(c3_hbm, c3)
    z16 = jnp.zeros((16,), jnp.float32)

    def zrow(r, cc):
        dba[r, :] = z16
        dbb[r, :] = z16
        return cc

    lax.fori_loop(0, CHG, zrow, 0)
    lane = lax.iota(jnp.int32, 16)

    def esl(j):
        return pl.ds((wid + j * NW) * CHG, CHG)

    def isl(j):
        return pl.ds(pl.multiple_of(j * CHG, 8), CHG)

    def compute(j, b1, b2, db):
        def addrow(r, cc):
            for q in range(D // 16):
                sl = pl.ds(q * 16, 16)
                b1[r, sl] = b1[r, sl] + b2[r, sl]
            return cc

        lax.fori_loop(0, CHG, addrow, 0)
        for k in range(CHG // 16):
            rows = k * 16 + lane
            off = pl.multiple_of(j * CHG + k * 16, 8)
            ir = idxr[pl.ds(off, 16)]
            ic = idxc[pl.ds(off, 16)]
            ir3 = ir * 3
            ic3 = ic * 3
            rad = z16
            for q in range(3):
                cq = jnp.full((16,), q, jnp.int32)
                xr = plsc.load_gather(c3, [ir3 + q])
                xc = plsc.load_gather(c3, [ic3 + q])
                dd = xr - xc
                rad = rad + dd * dd
                plsc.store_scatter(db, [rows, cq], dd)
            plsc.store_scatter(db, [rows, jnp.full((16,), 3, jnp.int32)], rad)
            rem = (ir & 7).astype(jnp.float32)
            plsc.store_scatter(db, [rows, jnp.full((16,), 4, jnp.int32)], rem)

    def step(j, b1, b2, db, ob1, ob2, odb):
        @pl.when(j >= 1)
        def _():
            pltpu.make_async_copy(ob1, g_hbm.at[esl(j - 1)], semw).wait()
            pltpu.make_async_copy(odb, d4_hbm.at[esl(j - 1)], semw).wait()

        @pl.when(j <= NFULLG - 2)
        def _():
            pltpu.async_copy(t1_hbm.at[idxr.at[isl(j + 1)]], ob1, semg)
            pltpu.async_copy(t2_hbm.at[idxc.at[isl(j + 1)]], ob2, semg)

        pltpu.make_async_copy(t1_hbm.at[idxr.at[isl(j)]], b1, semg).wait()
        pltpu.make_async_copy(t2_hbm.at[idxc.at[isl(j)]], b2, semg).wait()
        compute(j, b1, b2, db)
        pltpu.async_copy(b1, g_hbm.at[esl(j)], semw)
        pltpu.async_copy(db, d4_hbm.at[esl(j)], semw)

    pltpu.async_copy(t1_hbm.at[idxr.at[isl(0)]], b1a, semg)
    pltpu.async_copy(t2_hbm.at[idxc.at[isl(0)]], b2a, semg)

    def pair(m, cc):
        j = 2 * m
        step(j, b1a, b2a, dba, b1b, b2b, dbb)
        step(j + 1, b1b, b2b, dbb, b1a, b2a, dba)
        return cc

    lax.fori_loop(0, NFULLG // 2, pair, 0)
    pltpu.make_async_copy(b1b, g_hbm.at[esl(NFULLG - 1)], semw).wait()
    pltpu.make_async_copy(dbb, d4_hbm.at[esl(NFULLG - 1)], semw).wait()

    @pl.when(wid < NCHG - NFULLG * NW)
    def _():
        j = NFULLG
        pltpu.async_copy(t1_hbm.at[idxr.at[isl(j)]], b1a, semg).wait()
        pltpu.async_copy(t2_hbm.at[idxc.at[isl(j)]], b2a, semg).wait()
        compute(j, b1a, b2a, dba)
        pltpu.sync_copy(b1a, g_hbm.at[esl(j)])
        pltpu.sync_copy(dba, d4_hbm.at[esl(j)])


def _sc_gather(t1, t2, roww, colw, c3flat):
    mesh = plsc.VectorSubcoreMesh(core_axis_name="c", subcore_axis_name="s",
                                  num_cores=NC, num_subcores=NS)
    f = pl.kernel(
        _gather_body,
        out_type=(
            jax.ShapeDtypeStruct((E, D), jnp.float32),
            jax.ShapeDtypeStruct((E, CP), jnp.float32),
        ),
        mesh=mesh,
        compiler_params=pltpu.CompilerParams(needs_layout_passes=False),
        scratch_types=[
            pltpu.VMEM((EPWG,), jnp.int32),
            pltpu.VMEM((EPWG,), jnp.int32),
            pltpu.VMEM((CHG, D), jnp.float32),
            pltpu.VMEM((CHG, D), jnp.float32),
            pltpu.VMEM((CHG, CP), jnp.float32),
            pltpu.VMEM((CHG, D), jnp.float32),
            pltpu.VMEM((CHG, D), jnp.float32),
            pltpu.VMEM((CHG, CP), jnp.float32),
            pltpu.VMEM((3 * N,), jnp.float32),
            pltpu.SemaphoreType.DMA,
            pltpu.SemaphoreType.DMA,
        ],
    )
    return f(t1, t2, roww, colw, c3flat)


def _edge_body(g_ref, d4_ref, ea_ref, wr_ref, wet_ref, eb1_ref, w2t_ref,
               eb2_ref, c1t_ref, cb1_ref, cw2_ref, e_ref, pay_ref):
    pre = g_ref[...]
    d4 = d4_ref[...]
    radial = d4[:, 3:4]
    z = (pre + radial * wr_ref[...] + eb1_ref[...]
         + jnp.dot(ea_ref[...], wet_ref[...], preferred_element_type=jnp.float32))
    x = _silu(z)
    e = _silu(jnp.dot(x, w2t_ref[...], preferred_element_type=jnp.float32)
              + eb2_ref[...])
    u = _silu(jnp.dot(e, c1t_ref[...], preferred_element_type=jnp.float32)
              + cb1_ref[...])
    t = jnp.sum(u * cw2_ref[...], axis=1, keepdims=True)
    inv = t / (jnp.sqrt(radial) + 1e-8)
    lane16 = lax.broadcasted_iota(jnp.int32, d4.shape, 1)
    tr = (d4 * inv * jnp.where(lane16 < 3, 1.0, 0.0)
          + jnp.where(lane16 == 3, 1.0, 0.0))
    mi = d4[:, 4:5].astype(jnp.int32)
    lane128 = lax.broadcasted_iota(jnp.int32, pre.shape, 1)
    tr8 = jnp.concatenate([tr] * (D // CP), axis=1)
    pay = jnp.where(lane128 // CP == mi, tr8, 0.0)
    e_ref[...] = e
    pay_ref[...] = pay


def _tc_edge(g, d4, edge_attr, wr, wet, eb1, w2t, eb2, c1t, cb1, cw2):
    grid = E // EBLK
    return pl.pallas_call(
        _edge_body,
        grid=(grid,),
        in_specs=[
            pl.BlockSpec((EBLK, D), lambda i: (i, 0)),
            pl.BlockSpec((EBLK, CP), lambda i: (i, 0)),
            pl.BlockSpec((EBLK, DE), lambda i: (i, 0)),
            pl.BlockSpec((1, D), lambda i: (0, 0)),
            pl.BlockSpec((DE, D), lambda i: (0, 0)),
            pl.BlockSpec((1, D), lambda i: (0, 0)),
            pl.BlockSpec((D, D), lambda i: (0, 0)),
            pl.BlockSpec((1, D), lambda i: (0, 0)),
            pl.BlockSpec((D, D), lambda i: (0, 0)),
            pl.BlockSpec((1, D), lambda i: (0, 0)),
            pl.BlockSpec((1, D), lambda i: (0, 0)),
        ],
        out_specs=[
            pl.BlockSpec((EBLK, D), lambda i: (i, 0)),
            pl.BlockSpec((EBLK, D), lambda i: (i, 0)),
        ],
        out_shape=[
            jax.ShapeDtypeStruct((E, D), jnp.float32),
            jax.ShapeDtypeStruct((E, D), jnp.float32),
        ],
    )(g, d4, edge_attr, wr, wet, eb1, w2t, eb2, c1t, cb1, cw2)


def _scatter_body(e_hbm, pay_hbm, row_hbm, row8_hbm, z_hbm,
                  aggp_hbm, ntp_hbm, ira, i8a, eba, pba, irb, i8b, ebb, pbb,
                  agg_sh, nt_sh, semr, sems):
    c = lax.axis_index("c")
    s = lax.axis_index("s")
    wid = s * NC + c
    nb = s * NPT
    pltpu.sync_copy(z_hbm.at[pl.ds(nb, NPC)], agg_sh.at[pl.ds(nb, NPC)])
    pltpu.sync_copy(z_hbm.at[pl.ds(s * NPT8, NPT8)],
                    nt_sh.at[pl.ds(s * NPT8, NPT8)])
    plsc.subcore_barrier()

    def esl(j):
        return pl.ds((wid + j * NW) * CHS, CHS)

    def loads(j, ir, i8, eb, pb):
        pltpu.async_copy(row_hbm.at[esl(j)], ir, semr)
        pltpu.async_copy(row8_hbm.at[esl(j)], i8, semr)
        pltpu.async_copy(e_hbm.at[esl(j)], eb, semr)
        pltpu.async_copy(pay_hbm.at[esl(j)], pb, semr)

    def wait_loads(j, ir, i8, eb, pb):
        pltpu.make_async_copy(row_hbm.at[esl(j)], ir, semr).wait()
        pltpu.make_async_copy(row8_hbm.at[esl(j)], i8, semr).wait()
        pltpu.make_async_copy(e_hbm.at[esl(j)], eb, semr).wait()
        pltpu.make_async_copy(pay_hbm.at[esl(j)], pb, semr).wait()

    def step(j, ir, i8, eb, pb, oir, oi8, oeb, opb):
        @pl.when(j >= 1)
        def _():
            pltpu.make_async_copy(oeb, agg_sh.at[oir], sems).wait()
            pltpu.make_async_copy(opb, nt_sh.at[oi8], sems).wait()

        @pl.when(j <= NFULLS - 2)
        def _():
            loads(j + 1, oir, oi8, oeb, opb)

        wait_loads(j, ir, i8, eb, pb)
        pltpu.async_copy(eb, agg_sh.at[ir], sems, add=True)
        pltpu.async_copy(pb, nt_sh.at[i8], sems, add=True)

    loads(0, ira, i8a, eba, pba)

    def pair(m, cc):
        j = 2 * m
        step(j, ira, i8a, eba, pba, irb, i8b, ebb, pbb)
        step(j + 1, irb, i8b, ebb, pbb, ira, i8a, eba, pba)
        return cc

    lax.fori_loop(0, NFULLS // 2, pair, 0)
    pltpu.make_async_copy(ebb, agg_sh.at[irb], sems).wait()
    pltpu.make_async_copy(pbb, nt_sh.at[i8b], sems).wait()

    @pl.when(wid < NCHS - NFULLS * NW)
    def _():
        j = NFULLS
        loads(j, ira, i8a, eba, pba)
        wait_loads(j, ira, i8a, eba, pba)
        pltpu.sync_copy(eba, agg_sh.at[ira], add=True)
        pltpu.sync_copy(pba, nt_sh.at[i8a], add=True)

    plsc.subcore_barrier()
    pltpu.sync_copy(agg_sh.at[pl.ds(nb, NPC)],
                    aggp_hbm.at[c, pl.ds(nb, NPC)])
    pltpu.sync_copy(nt_sh.at[pl.ds(s * NPT8, NPT8)],
                    ntp_hbm.at[c, pl.ds(s * NPT8, NPT8)])


def _sc_scatter(e, pay, row, row8, z):
    mesh = plsc.VectorSubcoreMesh(core_axis_name="c", subcore_axis_name="s",
                                  num_cores=NC, num_subcores=NS)
    f = pl.kernel(
        _scatter_body,
        out_type=(
            jax.ShapeDtypeStruct((NC, N, D), jnp.float32),
            jax.ShapeDtypeStruct((NC, N8, D), jnp.float32),
        ),
        mesh=mesh,
        scratch_types=[
            pltpu.VMEM((CHS,), jnp.int32),
            pltpu.VMEM((CHS,), jnp.int32),
            pltpu.VMEM((CHS, D), jnp.float32),
            pltpu.VMEM((CHS, D), jnp.float32),
            pltpu.VMEM((CHS,), jnp.int32),
            pltpu.VMEM((CHS,), jnp.int32),
            pltpu.VMEM((CHS, D), jnp.float32),
            pltpu.VMEM((CHS, D), jnp.float32),
            pltpu.VMEM_SHARED((N, D), jnp.float32),
            pltpu.VMEM_SHARED((N8, D), jnp.float32),
            pltpu.SemaphoreType.DMA,
            pltpu.SemaphoreType.DMA,
        ],
    )
    return f(e, pay, row, row8, z)


def _node_body(h_ref, ap_ref, nt_ref, cp_ref, w1ht_ref, w1at_ref, nb1_ref,
               w2t_ref, nb2_ref, lnw_ref, lnb_ref, ho_ref, co_ref):
    hb = h_ref[...]
    agg = ap_ref[0] + ap_ref[1]
    st = nt_ref[0] + nt_ref[1]
    x = _silu(jnp.dot(hb, w1ht_ref[...], preferred_element_type=jnp.float32)
              + jnp.dot(agg, w1at_ref[...], preferred_element_type=jnp.float32)
              + nb1_ref[...])
    n = jnp.dot(x, w2t_ref[...], preferred_element_type=jnp.float32) + nb2_ref[...]
    ho = hb + n
    mu = jnp.mean(ho, axis=1, keepdims=True)
    var = jnp.mean((ho - mu) ** 2, axis=1, keepdims=True)
    ho_ref[...] = (ho - mu) / jnp.sqrt(var + 1e-5) * lnw_ref[...] + lnb_ref[...]
    den = st[:, 3:4]
    co_ref[...] = cp_ref[...] + st / (den + 1e-8)


def _tc_node(h, aggp, ntp, cpad, w1ht, w1at, nb1, w2t, nb2, lnw, lnb):
    grid = N // NBLK
    return pl.pallas_call(
        _node_body,
        grid=(grid,),
        in_specs=[
            pl.BlockSpec((NBLK, D), lambda i: (i, 0)),
            pl.BlockSpec((NC, NBLK, D), lambda i: (0, i, 0)),
            pl.BlockSpec((NC, NBLK, CP), lambda i: (0, i, 0)),
            pl.BlockSpec((NBLK, CP), lambda i: (i, 0)),
            pl.BlockSpec((D, D), lambda i: (0, 0)),
            pl.BlockSpec((D, D), lambda i: (0, 0)),
            pl.BlockSpec((1, D), lambda i: (0, 0)),
            pl.BlockSpec((D, D), lambda i: (0, 0)),
            pl.BlockSpec((1, D), lambda i: (0, 0)),
            pl.BlockSpec((1, D), lambda i: (0, 0)),
            pl.BlockSpec((1, D), lambda i: (0, 0)),
        ],
        out_specs=[
            pl.BlockSpec((NBLK, D), lambda i: (i, 0)),
            pl.BlockSpec((NBLK, CP), lambda i: (i, 0)),
        ],
        out_shape=[
            jax.ShapeDtypeStruct((N, D), jnp.float32),
            jax.ShapeDtypeStruct((N, CP), jnp.float32),
        ],
    )(h, aggp, ntp, cpad, w1ht, w1at, nb1, w2t, nb2, lnw, lnb)


def kernel(h, edge_index, coord, edge_attr, ew1, eb1, ew2, eb2,
           nw1, nb1, nw2, nb2, cw1, cb1, cw2, ln_w, ln_b):
    row = edge_index[0].astype(jnp.int32)
    col = edge_index[1].astype(jnp.int32)

    def wmajor(ix):
        ixp = jnp.pad(ix, (0, JMAXG * NW * CHG - E))
        return ixp.reshape(JMAXG, NW, CHG).transpose(1, 0, 2).reshape(-1)

    roww = wmajor(row)
    colw = wmajor(col)
    row8 = row >> 3
    cpad = jnp.pad(coord, ((0, 0), (0, CP - 3)))
    c3flat = coord.reshape(-1)

    wat = ew1[:, :D].T
    wbt = ew1[:, D:2 * D].T
    wr = ew1[:, 2 * D].reshape(1, D)
    wet = ew1[:, 2 * D + 1:].T
    w1ht = nw1[:, :D].T
    w1at = nw1[:, D:].T

    t1, t2 = _tc_pre(h, wat, wbt)
    g, d4 = _sc_gather(t1, t2, roww, colw, c3flat)
    e, pay = _tc_edge(g, d4, edge_attr, wr, wet, eb1.reshape(1, D), ew2.T,
                      eb2.reshape(1, D), cw1.T, cb1.reshape(1, D), cw2)
    z = jnp.zeros((N, D), jnp.float32)
    aggp, ntp = _sc_scatter(e, pay, row, row8, z)
    nt = ntp.reshape(NC, N8 * 8, CP)[:, :N, :]
    h_out, co = _tc_node(h, aggp, nt, cpad, w1ht, w1at, nb1.reshape(1, D),
                         nw2.T, nb2.reshape(1, D), ln_w.reshape(1, D),
                         ln_b.reshape(1, D))
    return (h_out, co[:, :3], e)
